# baseline (speedup 1.0000x reference)
import jax
import jax.numpy as jnp
from jax import lax
from jax.experimental import pallas as pl
from jax.experimental.pallas import tpu as pltpu
from jax.experimental.pallas import tpu_sc as plsc

D = 128
NM = 50000
NU = 100000
NW = 32
NCOL = D // 16
CH = 128

U_CHUNKS = NU // CH
U_PAIRS = 12
U_REM = NU - U_CHUNKS * CH
M_CHUNKS = NM // CH
M_PAIRS = 6
M_REM = NM - M_CHUNKS * CH


def _u_group_body(rows_ref, seg_ref):
    def body(g, carry):
        a = list(carry)
        segv = seg_ref[pl.ds(g * 16, 16)]
        for j in range(16):
            s = segv[j]
            r = g * 16 + j
            for k in range(NCOL):
                x = rows_ref[r, pl.ds(k * 16, 16)]
                a[k] = a[k] + x
                a[8 + k] = a[8 + k] + x * s
        return tuple(a)
    return body


def _m_row_body(rows_ref):
    def body(r, carry):
        a = list(carry)
        for k in range(NCOL):
            a[k] = a[k] + rows_ref[r, pl.ds(k * 16, 16)]
        return tuple(a)
    return body


def _sc_body(xm_hbm, xu_hbm, seg_hbm, bias_hbm, out_hbm,
             u_a, u_b, m_a, m_b, sg_a, sg_b, partial, biasv, stage, shared,
             ru_a, ru_b, su_a, su_b, rm_a, rm_b):
    cid = lax.axis_index("c")
    sid = lax.axis_index("s")
    wid = cid * 16 + sid

    u_bufs = (u_a, u_b)
    m_bufs = (m_a, m_b)
    sg_bufs = (sg_a, sg_b)
    ru_sems = (ru_a, ru_b)
    su_sems = (su_a, su_b)
    rm_sems = (rm_a, rm_b)

    def start_u(g, b):
        pltpu.async_copy(xu_hbm.at[pl.ds(g * CH, CH)], u_bufs[b], ru_sems[b])
        pltpu.async_copy(seg_hbm.at[pl.ds(g * CH, CH)], sg_bufs[b],
                         su_sems[b])

    def start_m(g, b):
        pltpu.async_copy(xm_hbm.at[pl.ds(g * CH, CH)], m_bufs[b], rm_sems[b])

    def wait_u(b):
        pltpu.make_async_copy(xu_hbm.at[pl.ds(0, CH)], u_bufs[b],
                              ru_sems[b]).wait()
        pltpu.make_async_copy(seg_hbm.at[pl.ds(0, CH)], sg_bufs[b],
                              su_sems[b]).wait()

    def wait_m(b):
        pltpu.make_async_copy(xm_hbm.at[pl.ds(0, CH)], m_bufs[b],
                              rm_sems[b]).wait()

    zero = jnp.zeros((16,), jnp.float32)
    z8 = (zero,) * 8
    z16 = (zero,) * 16

    start_u(wid, 0)
    start_u(wid + NW, 1)
    acc = z16

    def u_pair(p, carry):
        for b in range(2):
            k = 2 * p + b
            wait_u(b)
            carry = lax.fori_loop(0, CH // 16,
                                  _u_group_body(u_bufs[b], sg_bufs[b]), carry)

            @pl.when(k + 2 < 2 * U_PAIRS)
            def _():
                start_u(wid + (k + 2) * NW, b)
        return carry

    acc = lax.fori_loop(0, U_PAIRS, u_pair, acc)

    g_epi = wid + 2 * U_PAIRS * NW
    vf_epi = jnp.where(g_epi < U_CHUNKS, 1.0, 0.0).astype(jnp.float32)
    g_dma = jnp.minimum(g_epi, U_CHUNKS - 1)
    pltpu.async_copy(xu_hbm.at[pl.ds(g_dma * CH, CH)], u_a, ru_a)
    pltpu.async_copy(seg_hbm.at[pl.ds(g_dma * CH, CH)], sg_a, su_a)
    pltpu.async_copy(xu_hbm.at[pl.ds(U_CHUNKS * CH, U_REM)],
                     u_b.at[pl.ds(0, U_REM)], ru_b)
    pltpu.async_copy(seg_hbm.at[pl.ds(U_CHUNKS * CH, U_REM)],
                     sg_b.at[pl.ds(0, U_REM)], su_b)
    start_m(wid, 0)
    start_m(wid + NW, 1)

    wait_u(0)
    tmp = lax.fori_loop(0, CH // 16, _u_group_body(u_a, sg_a), z16)
    acc = tuple(acc[i] + tmp[i] * vf_epi for i in range(16))

    pltpu.make_async_copy(xu_hbm.at[pl.ds(0, U_REM)],
                          u_b.at[pl.ds(0, U_REM)], ru_b).wait()
    pltpu.make_async_copy(seg_hbm.at[pl.ds(0, U_REM)],
                          sg_b.at[pl.ds(0, U_REM)], su_b).wait()
    is_w0 = jnp.where(wid == 0, 1.0, 0.0).astype(jnp.float32)
    tmp = lax.fori_loop(0, U_REM // 16, _u_group_body(u_b, sg_b), z16)
    acc = tuple(acc[i] + tmp[i] * is_w0 for i in range(16))

    def m_pair(p, carry):
        for b in range(2):
            k = 2 * p + b
            wait_m(b)
            carry = lax.fori_loop(0, CH, _m_row_body(m_bufs[b]), carry)

            @pl.when(k + 2 < 2 * M_PAIRS)
            def _():
                start_m(wid + (k + 2) * NW, b)
        return carry

    accm = lax.fori_loop(0, M_PAIRS, m_pair, z8)

    gm_epi = wid + 2 * M_PAIRS * NW
    vf_mepi = jnp.where(gm_epi < M_CHUNKS, 1.0, 0.0).astype(jnp.float32)
    gm_dma = jnp.minimum(gm_epi, M_CHUNKS - 1)
    pltpu.async_copy(xm_hbm.at[pl.ds(gm_dma * CH, CH)], m_a, rm_a)
    pltpu.async_copy(xm_hbm.at[pl.ds(M_CHUNKS * CH, M_REM)],
                     m_b.at[pl.ds(0, M_REM)], rm_b)
    wait_m(0)
    tmp = lax.fori_loop(0, CH, _m_row_body(m_a), z8)
    accm = tuple(accm[i] + tmp[i] * vf_mepi for i in range(8))
    pltpu.make_async_copy(xm_hbm.at[pl.ds(0, M_REM)],
                          m_b.at[pl.ds(0, M_REM)], rm_b).wait()
    tmp = lax.fori_loop(0, M_REM, _m_row_body(m_b), z8)
    accm = tuple(accm[i] + tmp[i] * is_w0 for i in range(8))

    for k in range(NCOL):
        sl = pl.ds(k * 16, 16)
        partial[0, sl] = accm[k]
        partial[1, sl] = acc[k] - acc[8 + k]
        partial[2, sl] = acc[8 + k]

    @pl.when(wid == 0)
    def _():
        pltpu.sync_copy(bias_hbm, biasv)
        for r in range(3):
            for k in range(NCOL):
                sl = pl.ds(k * 16, 16)
                partial[r, sl] = partial[r, sl] + biasv[r, sl]

    pltpu.sync_copy(partial, shared.at[sid])
    plsc.subcore_barrier()

    @pl.when(sid == 0)
    def _():
        pltpu.sync_copy(shared, stage)
        for r in range(3):
            for k in range(NCOL):
                sl = pl.ds(k * 16, 16)
                v = stage[0, r, sl]
                for s in range(1, 16):
                    v = v + stage[s, r, sl]
                partial[r, sl] = v
        pltpu.sync_copy(partial, out_hbm.at[cid])


def _sc_call(X_mapped, X_unmapped, segf, bias_stack):
    mesh = plsc.VectorSubcoreMesh(core_axis_name="c", subcore_axis_name="s")
    f = pl.kernel(
        _sc_body,
        out_type=jax.ShapeDtypeStruct((2, 8, D), jnp.float32),
        mesh=mesh,
        scratch_types=[
            pltpu.VMEM((CH, D), jnp.float32),
            pltpu.VMEM((CH, D), jnp.float32),
            pltpu.VMEM((CH, D), jnp.float32),
            pltpu.VMEM((CH, D), jnp.float32),
            pltpu.VMEM((CH,), jnp.float32),
            pltpu.VMEM((CH,), jnp.float32),
            pltpu.VMEM((8, D), jnp.float32),
            pltpu.VMEM((8, D), jnp.float32),
            pltpu.VMEM((16, 8, D), jnp.float32),
            pltpu.VMEM_SHARED((16, 8, D), jnp.float32),
            pltpu.SemaphoreType.DMA,
            pltpu.SemaphoreType.DMA,
            pltpu.SemaphoreType.DMA,
            pltpu.SemaphoreType.DMA,
            pltpu.SemaphoreType.DMA,
            pltpu.SemaphoreType.DMA,
        ],
    )
    return f(X_mapped, X_unmapped, segf, bias_stack)


def _combine_body(p_ref, o_ref):
    o_ref[...] = p_ref[0, 0:3] + p_ref[1, 0:3]


def kernel(X_mapped, X_unmapped, segment_ids, X_map_bias, X_connected_bias,
           X_unconnected_bias):
    segf = segment_ids.astype(jnp.float32)
    bias_stack = jnp.pad(jnp.concatenate(
        [X_map_bias, X_connected_bias, X_unconnected_bias], axis=0),
        ((0, 5), (0, 0)))
    pair = _sc_call(X_mapped, X_unmapped, segf, bias_stack)
    out = pl.pallas_call(
        _combine_body,
        out_shape=jax.ShapeDtypeStruct((3, D), jnp.float32),
    )(pair)
    return out.reshape(-1)
